# baseline (speedup 1.0000x reference)
import functools

import jax
import jax.numpy as jnp
from jax import lax
from jax.experimental import pallas as pl
from jax.experimental.pallas import tpu as pltpu
from jax.experimental.pallas import tpu_sc as plsc

B = 16384
L = 50
D = 64
DO = 64
N = B * L
V = 1000000
VH = V // 2

NC = 2
NS = 16
NW = NC * NS
PER_W = N // NW
CHUNK = 512
NCH = PER_W // CHUNK

HB = 8192
SB = 2 * HB
NSB = V // SB
TAIL = V - NSB * SB
TAILA = 512
TAILB = TAIL - TAILA
TAILH = TAIL // 2
VL = (NSB + 1) * HB


def _proj_body(tail_ref, w_ref, b_ref, tab_ref, o_ref,
               lo_v, hi_v, lsem, hsem):
    i = pl.program_id(0)

    def start(j, buf):
        @pl.when(j < NSB)
        def _():
            pltpu.make_async_copy(tab_ref.at[:, pl.ds(j * SB, HB)],
                                  lo_v.at[buf], lsem.at[buf]).start()
            pltpu.make_async_copy(tab_ref.at[:, pl.ds(j * SB + HB, HB)],
                                  hi_v.at[buf], hsem.at[buf]).start()

        @pl.when(j == NSB)
        def _():
            pltpu.make_async_copy(tab_ref.at[:, pl.ds(NSB * SB, TAILA)],
                                  lo_v.at[buf, :, pl.ds(0, TAILA)],
                                  lsem.at[buf]).start()

    def wait(j, buf):
        @pl.when(j < NSB)
        def _():
            pltpu.make_async_copy(tab_ref.at[:, pl.ds(j * SB, HB)],
                                  lo_v.at[buf], lsem.at[buf]).wait()
            pltpu.make_async_copy(tab_ref.at[:, pl.ds(j * SB + HB, HB)],
                                  hi_v.at[buf], hsem.at[buf]).wait()

        @pl.when(j == NSB)
        def _():
            pltpu.make_async_copy(tab_ref.at[:, pl.ds(NSB * SB, TAILA)],
                                  lo_v.at[buf, :, pl.ds(0, TAILA)],
                                  lsem.at[buf]).wait()

    cur = lax.rem(i, 2)

    @pl.when(i == 0)
    def _():
        start(0, 0)

    wait(i, cur)
    start(i + 1, lax.rem(i + 1, 2))

    bb = b_ref[...]
    dn = (((0,), (0,)), ((), ()))

    @pl.when(i < NSB)
    def _():
        ylo = lax.dot_general(lo_v[cur], w_ref[...], dn,
                              preferred_element_type=jnp.float32)
        yhi = lax.dot_general(hi_v[cur], w_ref[...], dn,
                              preferred_element_type=jnp.float32)
        o_ref[:, 0:DO] = ylo + bb
        o_ref[:, DO:128] = yhi + bb

    @pl.when(i == NSB)
    def _():
        xa = lo_v[cur, :, 0:TAILA]
        ya = lax.dot_general(xa, w_ref[...], dn,
                             preferred_element_type=jnp.float32)
        yb = jnp.dot(tail_ref[...], w_ref[...],
                     preferred_element_type=jnp.float32)
        y = jnp.concatenate([ya, yb], axis=0) + bb
        o_ref[0:TAILH, 0:DO] = y[0:TAILH]
        o_ref[0:TAILH, DO:128] = y[TAILH:TAIL]


def _tc_project(tableT, tail64, Wt, b2):
    return pl.pallas_call(
        _proj_body,
        grid=(NSB + 1,),
        in_specs=[
            pl.BlockSpec((TAILB, D), lambda i: (0, 0)),
            pl.BlockSpec((D, DO), lambda i: (0, 0)),
            pl.BlockSpec((1, DO), lambda i: (0, 0)),
            pl.BlockSpec(memory_space=pl.ANY),
        ],
        out_specs=pl.BlockSpec((HB, 128), lambda i: (i, 0)),
        out_shape=jax.ShapeDtypeStruct((VL, 128), jnp.float32),
        scratch_shapes=[
            pltpu.VMEM((2, D, HB), jnp.float32),
            pltpu.VMEM((2, D, HB), jnp.float32),
            pltpu.SemaphoreType.DMA((2,)),
            pltpu.SemaphoreType.DMA((2,)),
        ],
        compiler_params=pltpu.CompilerParams(
            dimension_semantics=("arbitrary",),
        ),
    )(tail64, Wt, b2, tableT)


NP = 16
BP = B // NP
GB = 16
GCH = GB * L
B_PER_W = BP // NW
NCH2 = B_PER_W // GB


def _gather_body(idx_hbm, table_hbm, out_hbm, idx_v, rows_v, gsem, wsem):
    wid = lax.axis_index("s") * NC + lax.axis_index("c")
    b_base = wid * B_PER_W

    def idx_load(c, buf):
        pltpu.sync_copy(idx_hbm.at[pl.ds((b_base + c * GB) * L, GCH)],
                        idx_v.at[buf])

    def gather(c, buf):
        return pltpu.make_async_copy(table_hbm.at[idx_v.at[buf]],
                                     rows_v.at[buf], gsem.at[buf])

    def wb(c, buf, k):
        t = (b_base + c * GB) // 8 + (k // 8)
        u = k % 8
        g2 = t // 16
        r2 = lax.rem(t, 16)
        src = rows_v.at[buf].at[pl.ds(k * L, L), :]
        dst = out_hbm.at[g2 * 8 + lax.rem(r2, 8), :, u,
                         pl.ds((r2 // 8) * D, D)]
        return pltpu.make_async_copy(src, dst, wsem.at[buf])

    idx_load(0, 0)
    gather(0, 0).start()

    def step(c, _):
        cur = lax.rem(c, 2)
        nxt = lax.rem(c + 1, 2)

        gather(c, cur).wait()

        @pl.when(c + 1 < NCH2)
        def _():
            idx_load(c + 1, nxt)

        @pl.when(c >= 1)
        def _():
            for k in range(GB):
                wb(c - 1, nxt, k).wait()

        @pl.when(c + 1 < NCH2)
        def _():
            gather(c + 1, nxt).start()

        for k in range(GB):
            wb(c, cur, k).start()
        return ()

    lax.fori_loop(0, NCH2, step, ())
    for k in range(GB):
        wb(NCH2 - 1, lax.rem(NCH2 - 1, 2), k).wait()


def _sc_gather(idx, tableL):
    mesh = plsc.VectorSubcoreMesh(core_axis_name="c", subcore_axis_name="s")
    k = functools.partial(
        pl.kernel, mesh=mesh,
        out_type=jax.ShapeDtypeStruct((BP // 16, L, 8, 128), jnp.float32),
        scratch_types=[
            pltpu.VMEM((2, GCH), jnp.int32),
            pltpu.VMEM((2, GCH, D), jnp.float32),
            pltpu.SemaphoreType.DMA((2,)),
            pltpu.SemaphoreType.DMA((2,)),
        ],
        compiler_params=pltpu.CompilerParams(use_tc_tiling_on_sc=False),
    )(_gather_body)
    return k(idx, tableL)


TB = 8
PBLK = BP // (TB * 16)


def _tr_body(x_ref, o_ref):
    x = x_ref[...]
    for l in range(L):
        xl = x[:, l, :, :].reshape(TB * 8, 128)
        xt = xl.T
        o_ref[l * D:(l + 1) * D, 0:D] = xt[0:D]
        o_ref[l * D:(l + 1) * D, D:2 * D] = xt[D:128]


def _tr_body2(x_ref, zp_ref, o_ref):
    del zp_ref
    _tr_body(x_ref, o_ref)


def _tc_relayout_part(x4, z_prev, p):
    if p == 0:
        return pl.pallas_call(
            _tr_body,
            grid=(PBLK,),
            in_specs=[pl.BlockSpec((TB, L, 8, 128), lambda i: (i, 0, 0, 0))],
            out_specs=pl.BlockSpec((L * DO, TB * 16), lambda i: (0, i)),
            out_shape=jax.ShapeDtypeStruct((L * DO, B), jnp.float32),
            compiler_params=pltpu.CompilerParams(
                dimension_semantics=("arbitrary",),
            ),
        )(x4)
    return pl.pallas_call(
        _tr_body2,
        grid=(PBLK,),
        in_specs=[
            pl.BlockSpec((TB, L, 8, 128), lambda i: (i, 0, 0, 0)),
            pl.BlockSpec(memory_space=pl.ANY),
        ],
        out_specs=pl.BlockSpec((L * DO, TB * 16),
                               lambda i, _p=p: (0, _p * PBLK + i)),
        out_shape=jax.ShapeDtypeStruct((L * DO, B), jnp.float32),
        input_output_aliases={1: 0},
        compiler_params=pltpu.CompilerParams(
            dimension_semantics=("arbitrary",),
        ),
    )(x4, z_prev)


def kernel(src, table, W, b):
    idx = src.reshape(N).astype(jnp.int32)
    g = idx // SB
    r = idx % SB
    j_main = 2 * (g * HB + r % HB) + r // HB
    rr = idx - NSB * SB
    j_tail = 2 * (NSB * HB + rr % TAILH) + rr // TAILH
    idx2 = jnp.where(idx < NSB * SB, j_main, j_tail)
    tail64 = lax.slice(table, (V - TAILB, 0), (V, D))
    tP = _tc_project(table.T, tail64, W.T, b.reshape(1, DO))
    tL = tP.reshape(2 * VL, D)
    z2 = None
    NPART = N // NP
    for p in range(NP):
        idx_p = lax.slice(idx2, (p * NPART,), ((p + 1) * NPART,))
        x4_p = _sc_gather(idx_p, tL)
        z2 = _tc_relayout_part(x4_p, z2, p)
    return jnp.transpose(z2.reshape(L, DO, B), (2, 0, 1))


# speedup vs baseline: 1.0037x; 1.0037x over previous
import functools

import jax
import jax.numpy as jnp
from jax import lax
from jax.experimental import pallas as pl
from jax.experimental.pallas import tpu as pltpu
from jax.experimental.pallas import tpu_sc as plsc

B = 16384
L = 50
D = 64
DO = 64
N = B * L
V = 1000000
VH = V // 2

NC = 2
NS = 16
NW = NC * NS
PER_W = N // NW
CHUNK = 512
NCH = PER_W // CHUNK

HB = 8192
SB = 2 * HB
NSB = V // SB
TAIL = V - NSB * SB
TAILA = 512
TAILB = TAIL - TAILA
TAILH = TAIL // 2
VL = (NSB + 1) * HB


def _proj_body(tail_ref, w_ref, b_ref, tab_ref, o_ref,
               lo_v, hi_v, lsem, hsem):
    i = pl.program_id(0)

    def start(j, buf):
        @pl.when(j < NSB)
        def _():
            pltpu.make_async_copy(tab_ref.at[:, pl.ds(j * SB, HB)],
                                  lo_v.at[buf], lsem.at[buf]).start()
            pltpu.make_async_copy(tab_ref.at[:, pl.ds(j * SB + HB, HB)],
                                  hi_v.at[buf], hsem.at[buf]).start()

        @pl.when(j == NSB)
        def _():
            pltpu.make_async_copy(tab_ref.at[:, pl.ds(NSB * SB, TAILA)],
                                  lo_v.at[buf, :, pl.ds(0, TAILA)],
                                  lsem.at[buf]).start()

    def wait(j, buf):
        @pl.when(j < NSB)
        def _():
            pltpu.make_async_copy(tab_ref.at[:, pl.ds(j * SB, HB)],
                                  lo_v.at[buf], lsem.at[buf]).wait()
            pltpu.make_async_copy(tab_ref.at[:, pl.ds(j * SB + HB, HB)],
                                  hi_v.at[buf], hsem.at[buf]).wait()

        @pl.when(j == NSB)
        def _():
            pltpu.make_async_copy(tab_ref.at[:, pl.ds(NSB * SB, TAILA)],
                                  lo_v.at[buf, :, pl.ds(0, TAILA)],
                                  lsem.at[buf]).wait()

    cur = lax.rem(i, 2)

    @pl.when(i == 0)
    def _():
        start(0, 0)

    wait(i, cur)
    start(i + 1, lax.rem(i + 1, 2))

    bb = b_ref[...]
    dn = (((0,), (0,)), ((), ()))

    @pl.when(i < NSB)
    def _():
        ylo = lax.dot_general(lo_v[cur], w_ref[...], dn,
                              preferred_element_type=jnp.float32)
        yhi = lax.dot_general(hi_v[cur], w_ref[...], dn,
                              preferred_element_type=jnp.float32)
        o_ref[:, 0:DO] = ylo + bb
        o_ref[:, DO:128] = yhi + bb

    @pl.when(i == NSB)
    def _():
        xa = lo_v[cur, :, 0:TAILA]
        ya = lax.dot_general(xa, w_ref[...], dn,
                             preferred_element_type=jnp.float32)
        yb = jnp.dot(tail_ref[...], w_ref[...],
                     preferred_element_type=jnp.float32)
        y = jnp.concatenate([ya, yb], axis=0) + bb
        o_ref[0:TAILH, 0:DO] = y[0:TAILH]
        o_ref[0:TAILH, DO:128] = y[TAILH:TAIL]


def _tc_project(tableT, tail64, Wt, b2):
    return pl.pallas_call(
        _proj_body,
        grid=(NSB + 1,),
        in_specs=[
            pl.BlockSpec((TAILB, D), lambda i: (0, 0)),
            pl.BlockSpec((D, DO), lambda i: (0, 0)),
            pl.BlockSpec((1, DO), lambda i: (0, 0)),
            pl.BlockSpec(memory_space=pl.ANY),
        ],
        out_specs=pl.BlockSpec((HB, 128), lambda i: (i, 0)),
        out_shape=jax.ShapeDtypeStruct((VL, 128), jnp.float32),
        scratch_shapes=[
            pltpu.VMEM((2, D, HB), jnp.float32),
            pltpu.VMEM((2, D, HB), jnp.float32),
            pltpu.SemaphoreType.DMA((2,)),
            pltpu.SemaphoreType.DMA((2,)),
        ],
        compiler_params=pltpu.CompilerParams(
            dimension_semantics=("arbitrary",),
        ),
    )(tail64, Wt, b2, tableT)


NP = 8
BP = B // NP
GB = 16
GCH = GB * L
B_PER_W = BP // NW
NCH2 = B_PER_W // GB


def _gather_body(idx_hbm, table_hbm, out_hbm, idx_v, rows_v, gsem, wsem):
    wid = lax.axis_index("s") * NC + lax.axis_index("c")
    b_base = wid * B_PER_W

    def idx_load(c, buf):
        pltpu.sync_copy(idx_hbm.at[pl.ds((b_base + c * GB) * L, GCH)],
                        idx_v.at[buf])

    def gather(c, buf):
        return pltpu.make_async_copy(table_hbm.at[idx_v.at[buf]],
                                     rows_v.at[buf], gsem.at[buf])

    def wb(c, buf, k):
        t = (b_base + c * GB) // 8 + (k // 8)
        u = k % 8
        g2 = t // 16
        r2 = lax.rem(t, 16)
        src = rows_v.at[buf].at[pl.ds(k * L, L), :]
        dst = out_hbm.at[g2 * 8 + lax.rem(r2, 8), :, u,
                         pl.ds((r2 // 8) * D, D)]
        return pltpu.make_async_copy(src, dst, wsem.at[buf])

    idx_load(0, 0)
    gather(0, 0).start()

    def step(c, _):
        cur = lax.rem(c, 2)
        nxt = lax.rem(c + 1, 2)

        gather(c, cur).wait()

        @pl.when(c + 1 < NCH2)
        def _():
            idx_load(c + 1, nxt)

        @pl.when(c >= 1)
        def _():
            for k in range(GB):
                wb(c - 1, nxt, k).wait()

        @pl.when(c + 1 < NCH2)
        def _():
            gather(c + 1, nxt).start()

        for k in range(GB):
            wb(c, cur, k).start()
        return ()

    lax.fori_loop(0, NCH2, step, ())
    for k in range(GB):
        wb(NCH2 - 1, lax.rem(NCH2 - 1, 2), k).wait()


def _sc_gather(idx, tableL):
    mesh = plsc.VectorSubcoreMesh(core_axis_name="c", subcore_axis_name="s")
    k = functools.partial(
        pl.kernel, mesh=mesh,
        out_type=jax.ShapeDtypeStruct((BP // 16, L, 8, 128), jnp.float32),
        scratch_types=[
            pltpu.VMEM((2, GCH), jnp.int32),
            pltpu.VMEM((2, GCH, D), jnp.float32),
            pltpu.SemaphoreType.DMA((2,)),
            pltpu.SemaphoreType.DMA((2,)),
        ],
        compiler_params=pltpu.CompilerParams(use_tc_tiling_on_sc=False),
    )(_gather_body)
    return k(idx, tableL)


TB = 8
PBLK = BP // (TB * 16)


def _tr_body(x_ref, o_ref):
    x = x_ref[...]
    for l in range(L):
        xl = x[:, l, :, :].reshape(TB * 8, 128)
        xt = xl.T
        o_ref[l * D:(l + 1) * D, 0:D] = xt[0:D]
        o_ref[l * D:(l + 1) * D, D:2 * D] = xt[D:128]


def _tr_body2(x_ref, zp_ref, o_ref):
    del zp_ref
    _tr_body(x_ref, o_ref)


def _tc_relayout_part(x4, z_prev, p):
    if p == 0:
        return pl.pallas_call(
            _tr_body,
            grid=(PBLK,),
            in_specs=[pl.BlockSpec((TB, L, 8, 128), lambda i: (i, 0, 0, 0))],
            out_specs=pl.BlockSpec((L * DO, TB * 16), lambda i: (0, i)),
            out_shape=jax.ShapeDtypeStruct((L * DO, B), jnp.float32),
            compiler_params=pltpu.CompilerParams(
                dimension_semantics=("arbitrary",),
            ),
        )(x4)
    return pl.pallas_call(
        _tr_body2,
        grid=(PBLK,),
        in_specs=[
            pl.BlockSpec((TB, L, 8, 128), lambda i: (i, 0, 0, 0)),
            pl.BlockSpec(memory_space=pl.ANY),
        ],
        out_specs=pl.BlockSpec((L * DO, TB * 16),
                               lambda i, _p=p: (0, _p * PBLK + i)),
        out_shape=jax.ShapeDtypeStruct((L * DO, B), jnp.float32),
        input_output_aliases={1: 0},
        compiler_params=pltpu.CompilerParams(
            dimension_semantics=("arbitrary",),
        ),
    )(x4, z_prev)


def kernel(src, table, W, b):
    idx = src.reshape(N).astype(jnp.int32)
    g = idx // SB
    r = idx % SB
    j_main = 2 * (g * HB + r % HB) + r // HB
    rr = idx - NSB * SB
    j_tail = 2 * (NSB * HB + rr % TAILH) + rr // TAILH
    idx2 = jnp.where(idx < NSB * SB, j_main, j_tail)
    tail64 = lax.slice(table, (V - TAILB, 0), (V, D))
    tP = _tc_project(table.T, tail64, W.T, b.reshape(1, DO))
    tL = tP.reshape(2 * VL, D)
    z2 = None
    NPART = N // NP
    for p in range(NP):
        idx_p = lax.slice(idx2, (p * NPART,), ((p + 1) * NPART,))
        x4_p = _sc_gather(idx_p, tL)
        z2 = _tc_relayout_part(x4_p, z2, p)
    return jnp.transpose(z2.reshape(L, DO, B), (2, 0, 1))
